# baseline (speedup 1.0000x reference)
import functools
import jax
import jax.numpy as jnp
from jax import lax
from jax.experimental import pallas as pl
from jax.experimental.pallas import tpu as pltpu, tpu_sc as plsc

B, N, K, C, NS = 8, 16384, 16, 128, 4096
MAXC = N * K
IDX_PER_TILE = (B * N * K) // 32
QN = N // 4

_mesh = plsc.VectorSubcoreMesh(core_axis_name="c", subcore_axis_name="s")
_sc_params = pltpu.CompilerParams(needs_layout_passes=False)


U = 2
LUTS = 4096
ROWS_PT = IDX_PER_TILE // K
CH = 128


@functools.partial(
    pl.kernel,
    mesh=_mesh,
    out_type=jax.ShapeDtypeStruct((B * N,), jnp.float32),
    scratch_types=[
        pltpu.VMEM((CH, K), jnp.int32),
        pltpu.VMEM((CH, K), jnp.int32),
        pltpu.VMEM((U * N,), jnp.int32),
        pltpu.VMEM((QN,), jnp.int32),
        pltpu.VMEM((QN,), jnp.int32),
        pltpu.VMEM((QN,), jnp.float32),
        pltpu.VMEM((QN,), jnp.float32),
        pltpu.VMEM((LUTS,), jnp.float32),
        pltpu.VMEM_SHARED((16, N), jnp.int32),
        pltpu.SemaphoreType.DMA,
    ],
    compiler_params=_sc_params,
)
def _scores_sc(nid_hbm, lut_hbm, g_hbm, scores_out,
               nb0, nb1, hist_v, acc_v, tmp_v, lutv_v, g_v, lutsm_v, shared, sem):
    c = lax.axis_index("c")
    s = lax.axis_index("s")
    batch = c * 4 + s // 4
    quarter = s % 4
    rbase0 = quarter * ROWS_PT
    sbase = batch * N + quarter * QN

    @plsc.parallel_loop(0, U * N // 128, unroll=2)
    def _zero(i):
        for j in range(8):
            hist_v[pl.ds(i * 128 + j * 16, 16)] = jnp.zeros((16,), jnp.int32)

    ones = jnp.ones((16,), jnp.int32)
    bufs = (nb0, nb1)
    nch = ROWS_PT // CH
    handles = [None] * nch
    handles[0] = pltpu.async_copy(
        nid_hbm.at[batch, pl.ds(rbase0, CH), :], nb0, sem
    )
    for ch in range(nch):
        if ch + 1 < nch:
            handles[ch + 1] = pltpu.async_copy(
                nid_hbm.at[batch, pl.ds(rbase0 + (ch + 1) * CH, CH), :],
                bufs[(ch + 1) % 2], sem,
            )
        handles[ch].wait()
        cur = bufs[ch % 2]

        @plsc.parallel_loop(0, CH // U, unroll=2)
        def _scat(i):
            for j in range(U):
                iv = cur[i * U + j, :]
                plsc.addupdate_scatter(hist_v, [iv + j * N], ones)

    @plsc.parallel_loop(0, N // 128, unroll=2)
    def _red(i):
        for j in range(8):
            off = i * 128 + j * 16
            acc = hist_v[pl.ds(off, 16)]
            for u in range(1, U):
                acc = acc + hist_v[pl.ds(u * N + off, 16)]
            hist_v[pl.ds(off, 16)] = acc

    pltpu.sync_copy(hist_v.at[pl.ds(0, N)], shared.at[s])
    pltpu.sync_copy(lut_hbm.at[pl.ds(0, LUTS)], lutsm_v)
    plsc.subcore_barrier()

    group = s - quarter
    qoff = quarter * QN
    pltpu.sync_copy(shared.at[group, pl.ds(qoff, QN)], acc_v)
    for j in range(1, 4):
        pltpu.sync_copy(shared.at[group + j, pl.ds(qoff, QN)], tmp_v)

        @plsc.parallel_loop(0, QN // 128, unroll=2)
        def _add(i):
            for u in range(8):
                off = i * 128 + u * 16
                acc_v[pl.ds(off, 16)] = acc_v[pl.ds(off, 16)] + tmp_v[pl.ds(off, 16)]

    @plsc.parallel_loop(0, QN // 16, unroll=4, carry=jnp.int32(0))
    def _lookup(i, m):
        av = acc_v[pl.ds(i * 16, 16)]
        clamped = jnp.minimum(av, LUTS - 1)
        lutv_v[pl.ds(i * 16, 16)] = plsc.load_gather(lutsm_v, [clamped])
        return jnp.maximum(m, lax.reduce_max(av, (0,)))

    @pl.when(_lookup > LUTS - 1)
    def _slow():
        pltpu.async_copy(lut_hbm.at[acc_v], lutv_v, sem).wait()

    pltpu.sync_copy(g_hbm.at[pl.ds(sbase, QN)], g_v)

    @plsc.parallel_loop(0, QN // 128, unroll=2)
    def _score(i):
        for u in range(8):
            off = i * 128 + u * 16
            lutv_v[pl.ds(off, 16)] = lutv_v[pl.ds(off, 16)] + g_v[pl.ds(off, 16)]

    pltpu.sync_copy(lutv_v, scores_out.at[pl.ds(sbase, QN)])


def _cmpx(s, si, pos, d, k):
    is_lower = (pos & d) == 0
    bit_k = (pos & k) != 0
    ps = jnp.where(is_lower, jnp.roll(s, -d, axis=1), jnp.roll(s, d, axis=1))
    pi = jnp.where(is_lower, jnp.roll(si, -d, axis=1), jnp.roll(si, d, axis=1))
    g = (s > ps) | ((s == ps) & (si < pi))
    keep = g == (is_lower ^ bit_k)
    return jnp.where(keep, s, ps), jnp.where(keep, si, pi)


def _sort_kernel(s_ref, o_ref):
    s = s_ref[...]
    pos = lax.broadcasted_iota(jnp.int32, (B, N), 1)
    si = pos
    for k_log in range(1, 14):
        k = 1 << k_log
        for j_log in range(k_log - 1, -1, -1):
            s, si = _cmpx(s, si, pos, 1 << j_log, k)
    k = 1 << 14
    for j_log in range(13, -1, -1):
        d = 1 << j_log
        s, si = _cmpx(s, si, pos, d, k)
        if NS <= d <= N // 2:
            s, si, pos = s[:, :d], si[:, :d], pos[:, :d]
    o_ref[...] = si[:, :NS]


def _sort_tc(scores):
    return pl.pallas_call(
        _sort_kernel,
        out_shape=jax.ShapeDtypeStruct((B, NS), jnp.int32),
    )(scores)


RPT = (B * NS) // 32
FQ = RPT // 4


@functools.partial(
    pl.kernel,
    mesh=_mesh,
    out_type=[
        jax.ShapeDtypeStruct((B * NS,), jnp.float32),
        jax.ShapeDtypeStruct((B * NS,), jnp.float32),
        jax.ShapeDtypeStruct((B * NS,), jnp.float32),
        jax.ShapeDtypeStruct((B * NS, C), jnp.float32),
    ],
    scratch_types=[
        pltpu.VMEM((RPT,), jnp.int32),
        pltpu.VMEM((FQ, C), jnp.float32),
        pltpu.VMEM((FQ, C), jnp.float32),
        pltpu.VMEM((RPT,), jnp.float32),
        pltpu.VMEM((RPT,), jnp.float32),
        pltpu.VMEM((RPT,), jnp.float32),
        pltpu.SemaphoreType.DMA,
        pltpu.SemaphoreType.DMA,
        pltpu.SemaphoreType.DMA,
    ],
    compiler_params=_sc_params,
)
def _gather_sc(idxs_hbm, x_hbm, y_hbm, z_hbm, feat_hbm,
               x_out, y_out, z_out, feat_out,
               idx_v, fb0, fb1, xb, yb, zb, semf, semx, semo):
    c = lax.axis_index("c")
    s = lax.axis_index("s")
    w = s * 2 + c
    rbase = w * RPT
    b = w // 4
    pltpu.sync_copy(idxs_hbm.at[pl.ds(rbase, RPT)], idx_v)

    @plsc.parallel_loop(0, RPT // 16, unroll=2)
    def _base(i):
        idx_v[pl.ds(i * 16, 16)] = idx_v[pl.ds(i * 16, 16)] + b * N

    hx = pltpu.async_copy(x_hbm.at[idx_v], xb, semx)
    hy = pltpu.async_copy(y_hbm.at[idx_v], yb, semx)
    hz = pltpu.async_copy(z_hbm.at[idx_v], zb, semx)
    fb = (fb0, fb1)
    g = [None] * 4
    wrt = [None] * 4
    for q in range(2):
        g[q] = pltpu.async_copy(feat_hbm.at[idx_v.at[pl.ds(q * FQ, FQ)]], fb[q], semf)
    for q in range(4):
        g[q].wait()
        wrt[q] = pltpu.async_copy(fb[q % 2], feat_out.at[pl.ds(rbase + q * FQ, FQ)], semo)
        if q + 2 < 4:
            wrt[q].wait()
            g[q + 2] = pltpu.async_copy(
                feat_hbm.at[idx_v.at[pl.ds((q + 2) * FQ, FQ)]], fb[q % 2], semf
            )
    hx.wait()
    pltpu.sync_copy(xb, x_out.at[pl.ds(rbase, RPT)])
    hy.wait()
    pltpu.sync_copy(yb, y_out.at[pl.ds(rbase, RPT)])
    hz.wait()
    pltpu.sync_copy(zb, z_out.at[pl.ds(rbase, RPT)])
    wrt[2].wait()
    wrt[3].wait()


def kernel(xyz, feature, raw_relative_feature, neighbors_idx):
    g = jax.random.gumbel(jax.random.key(42), (B, N), dtype=jnp.float32)
    lut = jnp.log(1.0 / jnp.arange(MAXC + 1, dtype=jnp.float32))
    scores = _scores_sc(neighbors_idx, lut, g.reshape(-1))
    idxs = _sort_tc(scores.reshape(B, N))
    xyz_flat = xyz.reshape(B * N, 3)
    xo, yo, zo, new_feature = _gather_sc(
        idxs.reshape(-1),
        xyz_flat[:, 0], xyz_flat[:, 1], xyz_flat[:, 2],
        feature.reshape(B * N, C),
    )
    new_xyz = jnp.stack([xo, yo, zo], axis=-1).reshape(B, NS, 3)
    return new_xyz, new_feature.reshape(B, NS, C)


# speedup vs baseline: 1.0248x; 1.0248x over previous
import functools
import jax
import jax.numpy as jnp
from jax import lax
from jax.experimental import pallas as pl
from jax.experimental.pallas import tpu as pltpu, tpu_sc as plsc

B, N, K, C, NS = 8, 16384, 16, 128, 4096
MAXC = N * K
IDX_PER_TILE = (B * N * K) // 32
QN = N // 4

_mesh = plsc.VectorSubcoreMesh(core_axis_name="c", subcore_axis_name="s")
_sc_params = pltpu.CompilerParams(needs_layout_passes=False)


U = 2
LUTS = 4096
ROWS_PT = IDX_PER_TILE // K
CH = 128


@functools.partial(
    pl.kernel,
    mesh=_mesh,
    out_type=jax.ShapeDtypeStruct((B * N,), jnp.float32),
    scratch_types=[
        pltpu.VMEM((CH, K), jnp.int32),
        pltpu.VMEM((CH, K), jnp.int32),
        pltpu.VMEM((U * N,), jnp.int32),
        pltpu.VMEM((QN,), jnp.int32),
        pltpu.VMEM((QN,), jnp.int32),
        pltpu.VMEM((QN,), jnp.float32),
        pltpu.VMEM((LUTS,), jnp.float32),
        pltpu.VMEM_SHARED((16, N), jnp.int32),
        pltpu.SemaphoreType.DMA,
    ],
    compiler_params=_sc_params,
)
def _scores_sc(nid_hbm, lut_hbm, scores_out,
               nb0, nb1, hist_v, acc_v, tmp_v, lutv_v, lutsm_v, shared, sem):
    c = lax.axis_index("c")
    s = lax.axis_index("s")
    batch = c * 4 + s // 4
    quarter = s % 4
    rbase0 = quarter * ROWS_PT
    sbase = batch * N + quarter * QN

    @plsc.parallel_loop(0, U * N // 128, unroll=2)
    def _zero(i):
        for j in range(8):
            hist_v[pl.ds(i * 128 + j * 16, 16)] = jnp.zeros((16,), jnp.int32)

    ones = jnp.ones((16,), jnp.int32)
    bufs = (nb0, nb1)
    nch = ROWS_PT // CH
    handles = [None] * nch
    handles[0] = pltpu.async_copy(
        nid_hbm.at[batch, pl.ds(rbase0, CH), :], nb0, sem
    )
    for ch in range(nch):
        if ch + 1 < nch:
            handles[ch + 1] = pltpu.async_copy(
                nid_hbm.at[batch, pl.ds(rbase0 + (ch + 1) * CH, CH), :],
                bufs[(ch + 1) % 2], sem,
            )
        handles[ch].wait()
        cur = bufs[ch % 2]

        @plsc.parallel_loop(0, CH // U, unroll=2)
        def _scat(i):
            for j in range(U):
                iv = cur[i * U + j, :]
                plsc.addupdate_scatter(hist_v, [iv + j * N], ones)

    @plsc.parallel_loop(0, N // 128, unroll=2)
    def _red(i):
        for j in range(8):
            off = i * 128 + j * 16
            acc = hist_v[pl.ds(off, 16)]
            for u in range(1, U):
                acc = acc + hist_v[pl.ds(u * N + off, 16)]
            hist_v[pl.ds(off, 16)] = acc

    pltpu.sync_copy(hist_v.at[pl.ds(0, N)], shared.at[s])
    pltpu.sync_copy(lut_hbm.at[pl.ds(0, LUTS)], lutsm_v)
    plsc.subcore_barrier()

    group = s - quarter
    qoff = quarter * QN
    pltpu.sync_copy(shared.at[group, pl.ds(qoff, QN)], acc_v)
    for j in range(1, 4):
        pltpu.sync_copy(shared.at[group + j, pl.ds(qoff, QN)], tmp_v)

        @plsc.parallel_loop(0, QN // 128, unroll=2)
        def _add(i):
            for u in range(8):
                off = i * 128 + u * 16
                acc_v[pl.ds(off, 16)] = acc_v[pl.ds(off, 16)] + tmp_v[pl.ds(off, 16)]

    @plsc.parallel_loop(0, QN // 16, unroll=4, carry=jnp.int32(0))
    def _lookup(i, m):
        av = acc_v[pl.ds(i * 16, 16)]
        clamped = jnp.minimum(av, LUTS - 1)
        lutv_v[pl.ds(i * 16, 16)] = plsc.load_gather(lutsm_v, [clamped])
        return jnp.maximum(m, lax.reduce_max(av, (0,)))

    @pl.when(_lookup > LUTS - 1)
    def _slow():
        pltpu.async_copy(lut_hbm.at[acc_v], lutv_v, sem).wait()

    pltpu.sync_copy(lutv_v, scores_out.at[pl.ds(sbase, QN)])


def _cmpx(s, si, pos, d, k):
    is_lower = (pos & d) == 0
    bit_k = (pos & k) != 0
    ps = jnp.where(is_lower, jnp.roll(s, -d, axis=1), jnp.roll(s, d, axis=1))
    pi = jnp.where(is_lower, jnp.roll(si, -d, axis=1), jnp.roll(si, d, axis=1))
    g = (s > ps) | ((s == ps) & (si < pi))
    keep = g == (is_lower ^ bit_k)
    return jnp.where(keep, s, ps), jnp.where(keep, si, pi)


def _sort_kernel(s_ref, g_ref, o_ref):
    s = s_ref[...] + g_ref[...]
    pos = lax.broadcasted_iota(jnp.int32, (B, N), 1)
    si = pos
    for k_log in range(1, 14):
        k = 1 << k_log
        for j_log in range(k_log - 1, -1, -1):
            s, si = _cmpx(s, si, pos, 1 << j_log, k)
    k = 1 << 14
    for j_log in range(13, -1, -1):
        d = 1 << j_log
        s, si = _cmpx(s, si, pos, d, k)
        if NS <= d <= N // 2:
            s, si, pos = s[:, :d], si[:, :d], pos[:, :d]
    o_ref[...] = si[:, :NS]


def _sort_tc(scores, g):
    return pl.pallas_call(
        _sort_kernel,
        out_shape=jax.ShapeDtypeStruct((B, NS), jnp.int32),
    )(scores, g)


RPT = (B * NS) // 32
FQ = RPT // 4


@functools.partial(
    pl.kernel,
    mesh=_mesh,
    out_type=[
        jax.ShapeDtypeStruct((B * NS,), jnp.float32),
        jax.ShapeDtypeStruct((B * NS,), jnp.float32),
        jax.ShapeDtypeStruct((B * NS,), jnp.float32),
        jax.ShapeDtypeStruct((B * NS, C), jnp.float32),
    ],
    scratch_types=[
        pltpu.VMEM((RPT,), jnp.int32),
        pltpu.VMEM((FQ, C), jnp.float32),
        pltpu.VMEM((FQ, C), jnp.float32),
        pltpu.VMEM((RPT,), jnp.float32),
        pltpu.VMEM((RPT,), jnp.float32),
        pltpu.VMEM((RPT,), jnp.float32),
        pltpu.SemaphoreType.DMA,
        pltpu.SemaphoreType.DMA,
        pltpu.SemaphoreType.DMA,
    ],
    compiler_params=_sc_params,
)
def _gather_sc(idxs_hbm, x_hbm, y_hbm, z_hbm, feat_hbm,
               x_out, y_out, z_out, feat_out,
               idx_v, fb0, fb1, xb, yb, zb, semf, semx, semo):
    c = lax.axis_index("c")
    s = lax.axis_index("s")
    w = s * 2 + c
    rbase = w * RPT
    b = w // 4
    pltpu.sync_copy(idxs_hbm.at[pl.ds(rbase, RPT)], idx_v)

    @plsc.parallel_loop(0, RPT // 16, unroll=2)
    def _base(i):
        idx_v[pl.ds(i * 16, 16)] = idx_v[pl.ds(i * 16, 16)] + b * N

    hx = pltpu.async_copy(x_hbm.at[idx_v], xb, semx)
    hy = pltpu.async_copy(y_hbm.at[idx_v], yb, semx)
    hz = pltpu.async_copy(z_hbm.at[idx_v], zb, semx)
    fb = (fb0, fb1)
    g = [None] * 4
    wrt = [None] * 4
    for q in range(2):
        g[q] = pltpu.async_copy(feat_hbm.at[idx_v.at[pl.ds(q * FQ, FQ)]], fb[q], semf)
    for q in range(4):
        g[q].wait()
        wrt[q] = pltpu.async_copy(fb[q % 2], feat_out.at[pl.ds(rbase + q * FQ, FQ)], semo)
        if q + 2 < 4:
            wrt[q].wait()
            g[q + 2] = pltpu.async_copy(
                feat_hbm.at[idx_v.at[pl.ds((q + 2) * FQ, FQ)]], fb[q % 2], semf
            )
    hx.wait()
    pltpu.sync_copy(xb, x_out.at[pl.ds(rbase, RPT)])
    hy.wait()
    pltpu.sync_copy(yb, y_out.at[pl.ds(rbase, RPT)])
    hz.wait()
    pltpu.sync_copy(zb, z_out.at[pl.ds(rbase, RPT)])
    wrt[2].wait()
    wrt[3].wait()


def kernel(xyz, feature, raw_relative_feature, neighbors_idx):
    g = jax.random.gumbel(jax.random.key(42), (B, N), dtype=jnp.float32)
    lut = jnp.log(1.0 / jnp.arange(MAXC + 1, dtype=jnp.float32))
    scores = _scores_sc(neighbors_idx, lut)
    idxs = _sort_tc(scores.reshape(B, N), g)
    xyz_flat = xyz.reshape(B * N, 3)
    xo, yo, zo, new_feature = _gather_sc(
        idxs.reshape(-1),
        xyz_flat[:, 0], xyz_flat[:, 1], xyz_flat[:, 2],
        feature.reshape(B * N, C),
    )
    new_xyz = jnp.stack([xo, yo, zo], axis=-1).reshape(B, NS, 3)
    return new_xyz, new_feature.reshape(B, NS, C)
